# baseline (speedup 1.0000x reference)
import functools

import jax
import jax.numpy as jnp
from jax import lax
from jax.experimental import pallas as pl
from jax.experimental.pallas import tpu as pltpu
from jax.experimental.pallas import tpu_sc as plsc

N = 50000
N_MOD = 3
PROJ = 32
HID = 64
HALF = 32
E = 1600000
NC = 2
NS = 16
N_PAD = 51200
E_PAD = 1638400
ROWS = E_PAD // 128
ROWS_T = ROWS // NS
BLK_CONV = ROWS_T // 8
ROWS_W = ROWS // (NC * NS)
BLK_DEG = ROWS_W // 8
RPT = N_PAD // NS
DUMMY = N

_mesh = plsc.VectorSubcoreMesh(core_axis_name="c", subcore_axis_name="s")
_sc_params = pltpu.CompilerParams(needs_layout_passes=False,
                                  use_tc_tiling_on_sc=False)


@functools.partial(
    pl.kernel,
    out_type=jax.ShapeDtypeStruct((NC * NS, N_PAD), jnp.float32),
    mesh=_mesh,
    scratch_types=[
        pltpu.VMEM((8, 128), jnp.int32),
        pltpu.VMEM((8, 128), jnp.int32),
        pltpu.VMEM((N_PAD,), jnp.float32),
        pltpu.SemaphoreType.DMA,
    ],
    compiler_params=_sc_params,
)
def _deg(edges, out, didx0, didx1, deg, sem_i):
    c = lax.axis_index("c")
    s = lax.axis_index("s")
    w = c * NS + s
    zeros = jnp.zeros((16,), jnp.float32)

    @pl.loop(0, N_PAD // 16)
    def _zero(i):
        deg[pl.ds(i * 16, 16)] = zeros

    ones = jnp.ones((16,), jnp.float32)
    base = w * BLK_DEG * 8
    pltpu.async_copy(edges.at[1].at[pl.ds(base, 8)], didx0, sem_i)

    @pl.loop(0, BLK_DEG // 2)
    def _blk(b):
        for off, db, dbn in ((0, didx0, didx1), (1, didx1, didx0)):
            q = b * 2 + off
            row0 = base + q * 8
            pltpu.make_async_copy(edges.at[1].at[pl.ds(row0, 8)], db,
                                  sem_i).wait()

            @pl.when(q + 1 < BLK_DEG)
            def _prefetch():
                pltpu.async_copy(edges.at[1].at[pl.ds(row0 + 8, 8)], dbn,
                                 sem_i)

            for j in range(8):
                for k in range(8):
                    idx = db[j, pl.ds(k * 16, 16)]
                    plsc.addupdate_scatter(deg, [idx], ones)

    pltpu.sync_copy(deg, out.at[w])


DEPTH = 8
NPAIR = ROWS_T // DEPTH


@functools.partial(
    pl.kernel,
    out_type=jax.ShapeDtypeStruct((NC, N_PAD, HALF), jnp.bfloat16),
    mesh=_mesh,
    scratch_types=[
        pltpu.VMEM((2, DEPTH, 128), jnp.int32),
        pltpu.VMEM((2, DEPTH, 128), jnp.int32),
        pltpu.VMEM((DEPTH, 128), jnp.int32),
        pltpu.VMEM((DEPTH, 128, HALF), jnp.bfloat16),
        pltpu.VMEM_SHARED((N_PAD, HALF), jnp.bfloat16),
        pltpu.VMEM_SHARED((N_PAD, HALF), jnp.bfloat16),
        pltpu.SemaphoreType.DMA,
        pltpu.SemaphoreType.DMA,
        pltpu.SemaphoreType.DMA,
    ],
    compiler_params=_sc_params,
)
def _conv(g, edges, out, ib0, ib1, dbuf, bufs, acc, gtab, sem_i, sem_g,
          sem_s):
    c = lax.axis_index("c")
    s = lax.axis_index("s")
    base = s * ROWS_T

    @pl.loop(0, RPT // 128)
    def _init(i):
        r0 = s * RPT + i * 128
        pltpu.sync_copy(g.at[c].at[pl.ds(r0, 128)], bufs.at[0])
        pltpu.sync_copy(bufs.at[0], acc.at[pl.ds(r0, 128)])
        pltpu.sync_copy(bufs.at[0], gtab.at[pl.ds(r0, 128)])

    plsc.subcore_barrier()

    pltpu.async_copy(edges.at[0].at[pl.ds(base, DEPTH)], ib0.at[0], sem_i)
    pltpu.async_copy(edges.at[1].at[pl.ds(base, DEPTH)], ib0.at[1], sem_i)

    @pl.loop(0, NPAIR // 2)
    def _pair(p):
        for off, ib, ibn in ((0, ib0, ib1), (1, ib1, ib0)):
            q = p * 2 + off
            row0 = base + q * DEPTH
            pltpu.make_async_copy(edges.at[0].at[pl.ds(row0, DEPTH)],
                                  ib.at[0], sem_i).wait()
            pltpu.make_async_copy(edges.at[1].at[pl.ds(row0, DEPTH)],
                                  ib.at[1], sem_i).wait()

            @pl.when(q + 1 < NPAIR)
            def _prefetch():
                pltpu.async_copy(edges.at[0].at[pl.ds(row0 + DEPTH, DEPTH)],
                                 ibn.at[0], sem_i)
                pltpu.async_copy(edges.at[1].at[pl.ds(row0 + DEPTH, DEPTH)],
                                 ibn.at[1], sem_i)

            for j in range(DEPTH):
                @pl.when(q > 0)
                def _wait_prev_scatter():
                    pltpu.make_async_copy(bufs.at[j], acc.at[dbuf.at[j]],
                                          sem_s).wait()
                pltpu.async_copy(gtab.at[ib.at[0, j]], bufs.at[j], sem_g)
            for j in range(DEPTH):
                pltpu.make_async_copy(gtab.at[ib.at[0, j]], bufs.at[j],
                                      sem_g).wait()
                for k in range(8):
                    dbuf[j, pl.ds(k * 16, 16)] = ib[1, j, pl.ds(k * 16, 16)]
                pltpu.async_copy(bufs.at[j], acc.at[dbuf.at[j]], sem_s,
                                 add=True)

    for j in range(DEPTH):
        pltpu.make_async_copy(bufs.at[j], acc.at[dbuf.at[j]], sem_s).wait()

    plsc.subcore_barrier()

    @pl.loop(0, RPT // 128)
    def _wb(i):
        r0 = s * RPT + i * 128
        pltpu.sync_copy(acc.at[pl.ds(r0, 128)], bufs.at[0])
        pltpu.sync_copy(bufs.at[0], out.at[c].at[pl.ds(r0, 128)])


TBLK = 2048


def _dinv_of(degs_ref):
    deg = jnp.sum(degs_ref[...], axis=0) + 1.0
    return lax.rsqrt(deg)[:, None]


def _tc1_body(xp_ref, degs_ref, wp_ref, w1_ref, bp_ref, g_ref):
    dinv = _dinv_of(degs_ref)
    x = xp_ref[...]
    wp = wp_ref[...]
    nf = (x[:, 0:1] * wp[0:1, :] + x[:, 1:2] * wp[1:2, :]
          + x[:, 2:3] * wp[2:3, :] + bp_ref[...])
    hw = jnp.dot(nf, w1_ref[...], preferred_element_type=jnp.float32)
    gg = (dinv * hw).astype(jnp.bfloat16)
    g_ref[0] = gg[:, :HALF]
    g_ref[1] = gg[:, HALF:]


def _tc1(xp, degs, wp, w1, bp):
    nb = N_PAD // TBLK
    return pl.pallas_call(
        _tc1_body,
        grid=(nb,),
        in_specs=[
            pl.BlockSpec((TBLK, N_MOD), lambda i: (i, 0)),
            pl.BlockSpec((NC * NS, TBLK), lambda i: (0, i)),
            pl.BlockSpec((N_MOD, PROJ), lambda i: (0, 0)),
            pl.BlockSpec((PROJ, HID), lambda i: (0, 0)),
            pl.BlockSpec((1, PROJ), lambda i: (0, 0)),
        ],
        out_specs=pl.BlockSpec((NC, TBLK, HALF), lambda i: (0, i, 0)),
        out_shape=jax.ShapeDtypeStruct((NC, N_PAD, HALF), jnp.bfloat16),
    )(xp, degs, wp, w1, bp)


def _tc2_body(acc_ref, degs_ref, w2_ref, b1_ref, g2_ref):
    accb = jnp.concatenate([acc_ref[0], acc_ref[1]],
                           axis=1).astype(jnp.float32)
    dinv = _dinv_of(degs_ref)
    h1 = jnp.maximum(dinv * accb + b1_ref[...], 0.0)
    hw2 = jnp.dot(h1, w2_ref[...], preferred_element_type=jnp.float32)
    gg = (dinv * hw2).astype(jnp.bfloat16)
    g2_ref[0] = gg[:, :HALF]
    g2_ref[1] = gg[:, HALF:]


def _tc2(acc1, degs, w2, b1):
    nb = N_PAD // TBLK
    return pl.pallas_call(
        _tc2_body,
        grid=(nb,),
        in_specs=[
            pl.BlockSpec((NC, TBLK, HALF), lambda i: (0, i, 0)),
            pl.BlockSpec((NC * NS, TBLK), lambda i: (0, i)),
            pl.BlockSpec((HID, HID), lambda i: (0, 0)),
            pl.BlockSpec((1, HID), lambda i: (0, 0)),
        ],
        out_specs=pl.BlockSpec((NC, TBLK, HALF), lambda i: (0, i, 0)),
        out_shape=jax.ShapeDtypeStruct((NC, N_PAD, HALF), jnp.bfloat16),
    )(acc1, degs, w2, b1)


def _tc3_body(acc_ref, degs_ref, b2_ref, wc_ref, bc_ref, out_ref, sum_ref):
    i = pl.program_id(0)

    @pl.when(i == 0)
    def _():
        sum_ref[...] = jnp.zeros_like(sum_ref)

    accb = jnp.concatenate([acc_ref[0], acc_ref[1]],
                           axis=1).astype(jnp.float32)
    h2 = jnp.maximum(_dinv_of(degs_ref) * accb + b2_ref[...], 0.0)
    rows = i * TBLK + lax.broadcasted_iota(jnp.int32, (TBLK, 1), 0)
    h2 = jnp.where(rows < N, h2, 0.0)
    sum_ref[...] += jnp.sum(h2, axis=0, keepdims=True)

    @pl.when(i == pl.num_programs(0) - 1)
    def _():
        mean = sum_ref[...] * (1.0 / N)
        out_ref[...] = (jnp.dot(mean, wc_ref[...],
                                preferred_element_type=jnp.float32)
                        + bc_ref[...])


def _tc3(acc2, degs, b2, wc, bc):
    nb = N_PAD // TBLK
    return pl.pallas_call(
        _tc3_body,
        grid=(nb,),
        in_specs=[
            pl.BlockSpec((NC, TBLK, HALF), lambda i: (0, i, 0)),
            pl.BlockSpec((NC * NS, TBLK), lambda i: (0, i)),
            pl.BlockSpec((1, HID), lambda i: (0, 0)),
            pl.BlockSpec((HID, 1), lambda i: (0, 0)),
            pl.BlockSpec((1, 1), lambda i: (0, 0)),
        ],
        out_specs=pl.BlockSpec((1, 1), lambda i: (0, 0)),
        out_shape=jax.ShapeDtypeStruct((1, 1), jnp.float32),
        scratch_shapes=[pltpu.VMEM((1, HID), jnp.float32)],
    )(acc2, degs, b2, wc, bc)


@jax.jit
def _run(x, edge_index, W_proj, b_proj, W1, b1, W2, b2, Wc, bc):
    xr = x.reshape(N, N_MOD)
    xp = jnp.zeros((N_PAD, N_MOD), jnp.float32).at[:N].set(xr)
    ei = jnp.full((2, E_PAD), DUMMY, jnp.int32).at[:, :E].set(edge_index)
    edges = ei.reshape(2, ROWS, 128)

    degs = _deg(edges)
    g1 = _tc1(xp, degs, W_proj, W1, b_proj[None, :])
    acc1 = _conv(g1, edges)
    g2 = _tc2(acc1, degs, W2, b1[None, :])
    acc2 = _conv(g2, edges)
    return _tc3(acc2, degs, b2[None, :], Wc, bc[None, :])


def kernel(x, edge_index, W_proj, b_proj, W1, b1, W2, b2, Wc, bc):
    return _run(x, edge_index, W_proj, b_proj, W1, b1, W2, b2, Wc, bc)
